# Initial kernel scaffold; baseline (speedup 1.0000x reference)
import jax
import jax.numpy as jnp
from jax.experimental import pallas as pl


def kernel(x, edge_index, segment_ids, W1, Wp1, W2, Wp2, W3, Wp3, Wm1, bm1, Wm2, bm2, Wm3, bm3, Wd1, bd1, Wd2, bd2):
    raise NotImplementedError("write your pallas kernel here")


# speedup vs baseline: 9.2884x; 9.2884x over previous
import functools

import jax
import jax.numpy as jnp
from jax import lax
from jax.experimental import pallas as pl
from jax.experimental.pallas import tpu as pltpu
from jax.experimental.pallas import tpu_sc as plsc

N = 10000
E = 320000
G = 128

_TILES = 16
_RPT = 624
_TAIL0 = _TILES * _RPT
_TAILR = N - _TAIL0
_K = 80
_EPT = E // _TILES
_BN = 2000


def _copy_rows(src, dst, s):
    r0 = pl.multiple_of(s * _RPT, 8)
    pltpu.sync_copy(src.at[pl.ds(r0, _RPT)], dst.at[pl.ds(r0, _RPT)])

    @pl.when(s == _TILES - 1)
    def _():
        pltpu.sync_copy(src.at[pl.ds(_TAIL0, _TAILR)],
                        dst.at[pl.ds(_TAIL0, _TAILR)])

def _mesh():
    return plsc.VectorSubcoreMesh(core_axis_name="c", subcore_axis_name="s",
                                  num_cores=2, num_subcores=_TILES)


def _deg_body(dst_hbm, zeros_hbm, ones_hbm, d0_hbm, d1_hbm,
              idx_v, ones_v, acc_sh):
    c = lax.axis_index("c")
    s = lax.axis_index("s")
    ept = E // 2 // _TILES

    pltpu.sync_copy(ones_hbm, ones_v)
    _copy_rows(zeros_hbm, acc_sh, s)
    plsc.subcore_barrier()

    @pl.loop(0, ept // _K)
    def _(i):
        base = pl.multiple_of(c * (E // 2) + s * ept + i * _K, 8)
        pltpu.sync_copy(dst_hbm.at[pl.ds(base, _K)], idx_v)
        pltpu.sync_copy(ones_v, acc_sh.at[idx_v], add=True)

    plsc.subcore_barrier()

    @pl.when(c == 0)
    def _():
        _copy_rows(acc_sh, d0_hbm, s)

    @pl.when(c == 1)
    def _():
        _copy_rows(acc_sh, d1_hbm, s)


def _deg(dst, zeros128, ones128):
    return pl.kernel(
        _deg_body,
        out_type=(jax.ShapeDtypeStruct((N, 128), jnp.float32),
                  jax.ShapeDtypeStruct((N, 128), jnp.float32)),
        mesh=_mesh(),
        scratch_types=[
            pltpu.VMEM((_K,), jnp.int32),
            pltpu.VMEM((_K, 128), jnp.float32),
            pltpu.VMEM_SHARED((N, 128), jnp.float32),
        ],
    )(dst, zeros128, ones128)


def _dis_body(d0_ref, d1_ref, o_ref):
    o_ref[:, :] = lax.rsqrt(d0_ref[:, :16] + d1_ref[:, :16] + 1.0)


def _dis(d0, d1):
    return pl.pallas_call(
        _dis_body,
        grid=(N // _BN,),
        in_specs=[
            pl.BlockSpec((_BN, 128), lambda i: (i, 0)),
            pl.BlockSpec((_BN, 128), lambda i: (i, 0)),
        ],
        out_specs=pl.BlockSpec((_BN, 16), lambda i: (i, 0)),
        out_shape=jax.ShapeDtypeStruct((N, 16), jnp.float32),
    )(d0, d1)


def _agg_body(src_hbm, dst_hbm, hwa_hbm, hwb_hbm, zeros_hbm,
              outa_hbm, outb_hbm, isrc_v, idst_v, rows_v, sem, acc_sh):
    c = lax.axis_index("c")
    s = lax.axis_index("s")

    def work(tab, out):
        _copy_rows(zeros_hbm, acc_sh, s)
        plsc.subcore_barrier()

        @pl.loop(0, _EPT // _K)
        def _(i):
            base = pl.multiple_of(s * _EPT + i * _K, 8)
            pltpu.sync_copy(src_hbm.at[pl.ds(base, _K)], isrc_v)
            pltpu.sync_copy(dst_hbm.at[pl.ds(base, _K)], idst_v)
            pltpu.async_copy(tab.at[isrc_v], rows_v, sem).wait()
            pltpu.sync_copy(rows_v, acc_sh.at[idst_v], add=True)

        plsc.subcore_barrier()
        _copy_rows(acc_sh, out, s)

    @pl.when(c == 0)
    def _():
        work(hwa_hbm, outa_hbm)

    @pl.when(c == 1)
    def _():
        work(hwb_hbm, outb_hbm)


def _agg(src, dst, hwa, hwb, zeros, hh):
    return pl.kernel(
        _agg_body,
        out_type=(jax.ShapeDtypeStruct((N, hh), jnp.float32),
                  jax.ShapeDtypeStruct((N, hh), jnp.float32)),
        mesh=_mesh(),
        scratch_types=[
            pltpu.VMEM((_K,), jnp.int32),
            pltpu.VMEM((_K,), jnp.int32),
            pltpu.VMEM((_K, hh), jnp.float32),
            pltpu.SemaphoreType.DMA,
            pltpu.VMEM_SHARED((N, hh), jnp.float32),
        ],
    )(src, dst, hwa, hwb, zeros)


def _agg2_body(src_hbm, dst_hbm, hw_hbm, zeros_hbm,
               out0_hbm, out1_hbm, isrc_v, idst_v, rows_v, sem, acc_sh):
    c = lax.axis_index("c")
    s = lax.axis_index("s")
    ept = E // 2 // _TILES

    _copy_rows(zeros_hbm, acc_sh, s)
    plsc.subcore_barrier()

    @pl.loop(0, ept // _K)
    def _(i):
        base = pl.multiple_of((c * (E // 2)) + s * ept + i * _K, 8)
        pltpu.sync_copy(src_hbm.at[pl.ds(base, _K)], isrc_v)
        pltpu.sync_copy(dst_hbm.at[pl.ds(base, _K)], idst_v)
        pltpu.async_copy(hw_hbm.at[isrc_v], rows_v, sem).wait()
        pltpu.sync_copy(rows_v, acc_sh.at[idst_v], add=True)

    plsc.subcore_barrier()

    @pl.when(c == 0)
    def _():
        _copy_rows(acc_sh, out0_hbm, s)

    @pl.when(c == 1)
    def _():
        _copy_rows(acc_sh, out1_hbm, s)


def _agg2(src, dst, hw, zeros):
    return pl.kernel(
        _agg2_body,
        out_type=(jax.ShapeDtypeStruct((N, 128), jnp.float32),
                  jax.ShapeDtypeStruct((N, 128), jnp.float32)),
        mesh=_mesh(),
        scratch_types=[
            pltpu.VMEM((_K,), jnp.int32),
            pltpu.VMEM((_K,), jnp.int32),
            pltpu.VMEM((_K, 128), jnp.float32),
            pltpu.SemaphoreType.DMA,
            pltpu.VMEM_SHARED((N, 128), jnp.float32),
        ],
    )(src, dst, hw, zeros)


def _mm_body(h_ref, w_ref, dis_ref, a_ref, b_ref, *, hh):
    dis = dis_ref[:, 0:1]
    hw = jnp.dot(h_ref[:, :], w_ref[:, :], preferred_element_type=jnp.float32) * dis
    a_ref[:, :] = hw[:, :hh]
    b_ref[:, :] = hw[:, hh:]


def _mm(h, w, deg):
    din = h.shape[1]
    hfull = w.shape[1]
    hh = hfull // 2
    return pl.pallas_call(
        functools.partial(_mm_body, hh=hh),
        grid=(N // _BN,),
        in_specs=[
            pl.BlockSpec((_BN, din), lambda i: (i, 0)),
            pl.BlockSpec((din, hfull), lambda i: (0, 0)),
            pl.BlockSpec((_BN, 16), lambda i: (i, 0)),
        ],
        out_specs=[
            pl.BlockSpec((_BN, hh), lambda i: (i, 0)),
            pl.BlockSpec((_BN, hh), lambda i: (i, 0)),
        ],
        out_shape=[jax.ShapeDtypeStruct((N, hh), jnp.float32)] * 2,
    )(h, w, deg)


def _mmf_body(h_ref, w_ref, dis_ref, o_ref):
    dis = dis_ref[:, 0:1]
    o_ref[:, :] = jnp.dot(h_ref[:, :], w_ref[:, :],
                          preferred_element_type=jnp.float32) * dis


def _mmf(h, w, deg):
    din = h.shape[1]
    hfull = w.shape[1]
    return pl.pallas_call(
        _mmf_body,
        grid=(N // _BN,),
        in_specs=[
            pl.BlockSpec((_BN, din), lambda i: (i, 0)),
            pl.BlockSpec((din, hfull), lambda i: (0, 0)),
            pl.BlockSpec((_BN, 16), lambda i: (i, 0)),
        ],
        out_specs=pl.BlockSpec((_BN, hfull), lambda i: (i, 0)),
        out_shape=jax.ShapeDtypeStruct((N, hfull), jnp.float32),
    )(h, w, deg)


def _post_body(aa_ref, ab_ref, ha_ref, hb_ref, dis_ref, seg_ref,
               h_ref, summ_ref, cnt_ref):
    i = pl.program_id(0)
    dis = dis_ref[:, 0:1]
    left = (aa_ref[:, :] + ha_ref[:, :]) * dis
    right = (ab_ref[:, :] + hb_ref[:, :]) * dis
    h = jnp.maximum(jnp.concatenate([left, right], axis=1), 0.0)
    h_ref[:, :] = h
    onehot = (seg_ref[:, :] == lax.broadcasted_iota(jnp.int32, (_BN, G), 1)
              ).astype(jnp.float32)
    psum = lax.dot_general(onehot, h, (((0,), (0,)), ((), ())),
                           preferred_element_type=jnp.float32)
    pcnt = lax.dot_general(onehot, jnp.ones((_BN, 128), jnp.float32),
                           (((0,), (0,)), ((), ())),
                           preferred_element_type=jnp.float32)

    @pl.when(i == 0)
    def _():
        summ_ref[:, :] = jnp.zeros_like(summ_ref)
        cnt_ref[:, :] = jnp.zeros_like(cnt_ref)

    summ_ref[:, :] += psum
    cnt_ref[:, :] += pcnt


def _post2_body(p0_ref, p1_ref, hw_ref, dis_ref, seg_ref,
                h_ref, summ_ref, cnt_ref):
    i = pl.program_id(0)
    dis = dis_ref[:, 0:1]
    h = jnp.maximum((p0_ref[:, :] + p1_ref[:, :] + hw_ref[:, :]) * dis, 0.0)
    h_ref[:, :] = h
    onehot = (seg_ref[:, :] == lax.broadcasted_iota(jnp.int32, (_BN, G), 1)
              ).astype(jnp.float32)
    psum = lax.dot_general(onehot, h, (((0,), (0,)), ((), ())),
                           preferred_element_type=jnp.float32)
    pcnt = lax.dot_general(onehot, jnp.ones((_BN, 128), jnp.float32),
                           (((0,), (0,)), ((), ())),
                           preferred_element_type=jnp.float32)

    @pl.when(i == 0)
    def _():
        summ_ref[:, :] = jnp.zeros_like(summ_ref)
        cnt_ref[:, :] = jnp.zeros_like(cnt_ref)

    summ_ref[:, :] += psum
    cnt_ref[:, :] += pcnt


def _post2(p0, p1, hw, deg, segr):
    hfull = hw.shape[1]
    return pl.pallas_call(
        _post2_body,
        grid=(N // _BN,),
        in_specs=[
            pl.BlockSpec((_BN, hfull), lambda i: (i, 0)),
            pl.BlockSpec((_BN, hfull), lambda i: (i, 0)),
            pl.BlockSpec((_BN, hfull), lambda i: (i, 0)),
            pl.BlockSpec((_BN, 16), lambda i: (i, 0)),
            pl.BlockSpec((_BN, G), lambda i: (i, 0)),
        ],
        out_specs=[
            pl.BlockSpec((_BN, hfull), lambda i: (i, 0)),
            pl.BlockSpec((G, hfull), lambda i: (0, 0)),
            pl.BlockSpec((G, 128), lambda i: (0, 0)),
        ],
        out_shape=[
            jax.ShapeDtypeStruct((N, hfull), jnp.float32),
            jax.ShapeDtypeStruct((G, hfull), jnp.float32),
            jax.ShapeDtypeStruct((G, 128), jnp.float32),
        ],
    )(p0, p1, hw, deg, segr)


def _post(aa, ab, ha, hb, deg, segr):
    hh = aa.shape[1]
    hfull = 2 * hh
    return pl.pallas_call(
        _post_body,
        grid=(N // _BN,),
        in_specs=[
            pl.BlockSpec((_BN, hh), lambda i: (i, 0)),
            pl.BlockSpec((_BN, hh), lambda i: (i, 0)),
            pl.BlockSpec((_BN, hh), lambda i: (i, 0)),
            pl.BlockSpec((_BN, hh), lambda i: (i, 0)),
            pl.BlockSpec((_BN, 16), lambda i: (i, 0)),
            pl.BlockSpec((_BN, G), lambda i: (i, 0)),
        ],
        out_specs=[
            pl.BlockSpec((_BN, hfull), lambda i: (i, 0)),
            pl.BlockSpec((G, hfull), lambda i: (0, 0)),
            pl.BlockSpec((G, 128), lambda i: (0, 0)),
        ],
        out_shape=[
            jax.ShapeDtypeStruct((N, hfull), jnp.float32),
            jax.ShapeDtypeStruct((G, hfull), jnp.float32),
            jax.ShapeDtypeStruct((G, 128), jnp.float32),
        ],
    )(aa, ab, ha, hb, deg, segr)


def _pool_body(h_ref, seg_ref, summ_ref, cnt_ref, wp_ref, g_ref):
    i = pl.program_id(0)
    mean = summ_ref[:, :] / jnp.maximum(cnt_ref[:, 0:1], 1.0)
    cmat = jnp.tanh(jnp.dot(mean, wp_ref[:, :],
                            preferred_element_type=jnp.float32))
    onehot = (seg_ref[:, :] == lax.broadcasted_iota(jnp.int32, (_BN, G), 1)
              ).astype(jnp.float32)
    cnode = jnp.dot(onehot, cmat, preferred_element_type=jnp.float32)
    h = h_ref[:, :]
    att = 1.0 / (1.0 + jnp.exp(-jnp.sum(h * cnode, axis=1, keepdims=True)))

    @pl.when(i == 0)
    def _():
        g_ref[:, :] = jnp.zeros_like(g_ref)

    g_ref[:, :] += lax.dot_general(onehot, att * h, (((0,), (0,)), ((), ())),
                                   preferred_element_type=jnp.float32)


def _pool(h, segr, summ, cnt, wp):
    hfull = h.shape[1]
    return pl.pallas_call(
        _pool_body,
        grid=(N // _BN,),
        in_specs=[
            pl.BlockSpec((_BN, hfull), lambda i: (i, 0)),
            pl.BlockSpec((_BN, G), lambda i: (i, 0)),
            pl.BlockSpec((G, hfull), lambda i: (0, 0)),
            pl.BlockSpec((G, 128), lambda i: (0, 0)),
            pl.BlockSpec((hfull, hfull), lambda i: (0, 0)),
        ],
        out_specs=pl.BlockSpec((G, hfull), lambda i: (0, 0)),
        out_shape=jax.ShapeDtypeStruct((G, hfull), jnp.float32),
    )(h, segr, summ, cnt, wp)


def _head_body(g1_ref, g2_ref, g3_ref, w1a_ref, w1b_ref, w1c_ref, b1_ref,
               w2_ref, b2_ref, w3_ref, b3_ref, wd1_ref, bd1_ref,
               wd2_ref, bd2_ref, out_ref):
    m = (jnp.dot(g1_ref[:, :], w1a_ref[:, :], preferred_element_type=jnp.float32)
         + jnp.dot(g2_ref[:, :], w1b_ref[:, :], preferred_element_type=jnp.float32)
         + jnp.dot(g3_ref[:, :], w1c_ref[:, :], preferred_element_type=jnp.float32)
         + b1_ref[:, :])
    m = jnp.maximum(m, 0.0)
    m = jnp.maximum(jnp.dot(m, w2_ref[:, :], preferred_element_type=jnp.float32)
                    + b2_ref[:, :], 0.0)
    emb = jnp.dot(m, w3_ref[:, :], preferred_element_type=jnp.float32) + b3_ref[:, :]
    d = jnp.maximum(jnp.dot(emb, wd1_ref[:, :], preferred_element_type=jnp.float32)
                    + bd1_ref[:, :], 0.0)
    out_ref[:, :] = 0.5 * jnp.tanh(
        jnp.dot(d, wd2_ref[:, :], preferred_element_type=jnp.float32)
        + bd2_ref[:, :])


def _head(g1, g2, g3, w1a, w1b, w1c, bm1, wm2, bm2, wm3, bm3,
          wd1, bd1, wd2, bd2):
    return pl.pallas_call(
        _head_body,
        out_shape=jax.ShapeDtypeStruct((G, wd2.shape[1]), jnp.float32),
    )(g1, g2, g3, w1a, w1b, w1c,
      bm1[None, :], wm2, bm2[None, :], wm3, bm3[None, :],
      wd1, bd1[None, :], wd2, bd2[None, :])


def kernel(x, edge_index, segment_ids, W1, Wp1, W2, Wp2, W3, Wp3,
           Wm1, bm1, Wm2, bm2, Wm3, bm3, Wd1, bd1, Wd2, bd2):
    src = edge_index[0].astype(jnp.int32)
    dst = edge_index[1].astype(jnp.int32)
    segr = jnp.broadcast_to(segment_ids.astype(jnp.int32)[:, None], (N, G))

    zeros128 = jnp.zeros((N, 128), jnp.float32)
    ones128 = jnp.ones((_K, 128), jnp.float32)

    W3p = jnp.concatenate([W3, jnp.zeros((W3.shape[0], 64), jnp.float32)], 1)
    Wp3p = jnp.pad(Wp3, ((0, 64), (0, 64)))
    w1c = jnp.concatenate([Wm1[384:], jnp.zeros((64, Wm1.shape[1]),
                                                jnp.float32)], 0)

    d0, d1 = _deg(dst, zeros128, ones128)
    deg = _dis(d0, d1)

    hwa, hwb = _mm(x, W1, deg)
    aga, agb = _agg(src, dst, hwa, hwb, zeros128, 128)
    h1, summ1, cnt = _post(aga, agb, hwa, hwb, deg, segr)
    g1 = _pool(h1, segr, summ1, cnt, Wp1)

    hw2 = _mmf(h1, W2, deg)
    p0, p1 = _agg2(src, dst, hw2, zeros128)
    h2, summ2, cnt2 = _post2(p0, p1, hw2, deg, segr)
    g2 = _pool(h2, segr, summ2, cnt2, Wp2)

    hw3 = _mmf(h2, W3p, deg)
    p0, p1 = _agg2(src, dst, hw3, zeros128)
    h3, summ3, cnt3 = _post2(p0, p1, hw3, deg, segr)
    g3 = _pool(h3, segr, summ3, cnt3, Wp3p)

    return _head(g1, g2, g3, Wm1[:256], Wm1[256:384], w1c, bm1,
                 Wm2, bm2, Wm3, bm3, Wd1, bd1, Wd2, bd2)


# speedup vs baseline: 21.7925x; 2.3462x over previous
import functools

import jax
import jax.numpy as jnp
from jax import lax
from jax.experimental import pallas as pl
from jax.experimental.pallas import tpu as pltpu
from jax.experimental.pallas import tpu_sc as plsc

N = 10000
E = 320000
G = 128

_TILES = 16
_RPT = 624
_TAIL0 = _TILES * _RPT
_TAILR = N - _TAIL0
_K = 125
_NCC = E // _TILES // _K
_NCE = E // (2 * _TILES) // _K
_BN = 2000


def _copy_rows(src, dst, s):
    r0 = pl.multiple_of(s * _RPT, 8)
    pltpu.sync_copy(src.at[pl.ds(r0, _RPT)], dst.at[pl.ds(r0, _RPT)])

    @pl.when(s == _TILES - 1)
    def _():
        pltpu.sync_copy(src.at[pl.ds(_TAIL0, _TAILR)],
                        dst.at[pl.ds(_TAIL0, _TAILR)])

def _mesh():
    return plsc.VectorSubcoreMesh(core_axis_name="c", subcore_axis_name="s",
                                  num_cores=2, num_subcores=_TILES)


def _deg_body(dst3_hbm, zeros_hbm, ones_hbm, d0_hbm, d1_hbm,
              idst_all, ones_v, sem, acc_sh):
    c = lax.axis_index("c")
    s = lax.axis_index("s")
    w = c * _TILES + s

    pltpu.sync_copy(ones_hbm, ones_v)
    pltpu.sync_copy(dst3_hbm.at[w], idst_all)
    _copy_rows(zeros_hbm, acc_sh, s)
    plsc.subcore_barrier()

    @pl.loop(0, _NCE, step=8)
    def _(j):
        cps = [pltpu.async_copy(ones_v, acc_sh.at[idst_all.at[j + t]],
                                sem, add=True) for t in range(8)]
        for cp in cps:
            cp.wait()

    plsc.subcore_barrier()

    @pl.when(c == 0)
    def _():
        _copy_rows(acc_sh, d0_hbm, s)

    @pl.when(c == 1)
    def _():
        _copy_rows(acc_sh, d1_hbm, s)


def _deg(dst3, zeros128, ones128):
    return pl.kernel(
        _deg_body,
        out_type=(jax.ShapeDtypeStruct((N, 128), jnp.float32),
                  jax.ShapeDtypeStruct((N, 128), jnp.float32)),
        mesh=_mesh(),
        scratch_types=[
            pltpu.VMEM((_NCE, _K), jnp.int32),
            pltpu.VMEM((_K, 128), jnp.float32),
            pltpu.SemaphoreType.DMA,
            pltpu.VMEM_SHARED((N, 128), jnp.float32),
        ],
    )(dst3, zeros128, ones128)


def _dis_body(d0_ref, d1_ref, o_ref):
    o_ref[:, :] = lax.rsqrt(d0_ref[:, :16] + d1_ref[:, :16] + 1.0)


def _dis(d0, d1):
    return pl.pallas_call(
        _dis_body,
        grid=(N // _BN,),
        in_specs=[
            pl.BlockSpec((_BN, 128), lambda i: (i, 0)),
            pl.BlockSpec((_BN, 128), lambda i: (i, 0)),
        ],
        out_specs=pl.BlockSpec((_BN, 16), lambda i: (i, 0)),
        out_shape=jax.ShapeDtypeStruct((N, 16), jnp.float32),
    )(d0, d1)


def _edge_pipeline(tab, acc_sh, src3, dst3, w, nc, isl, idl, rows, semi, semr):
    for q in range(4):
        pltpu.async_copy(src3.at[w, q], isl[q], semi[q])
        pltpu.async_copy(dst3.at[w, q], idl[q], semi[q])
    for i in range(2):
        pltpu.make_async_copy(src3.at[w, 0], isl[i], semi[i]).wait()
        pltpu.make_async_copy(src3.at[w, 0], idl[i], semi[i]).wait()
        pltpu.async_copy(tab.at[isl[i]], rows[i], semr[i])

    @pl.loop(0, nc - 4, step=4)
    def _(j):
        for b in range(4):
            p, q, q2 = b % 2, b, (b + 2) % 4
            pltpu.make_async_copy(tab.at[isl[0]], rows[p], semr[p]).wait()
            pltpu.sync_copy(rows[p], acc_sh.at[idl[q]], add=True)
            pltpu.async_copy(src3.at[w, j + b + 4], isl[q], semi[q])
            pltpu.async_copy(dst3.at[w, j + b + 4], idl[q], semi[q])
            pltpu.make_async_copy(src3.at[w, 0], isl[q2], semi[q2]).wait()
            pltpu.make_async_copy(src3.at[w, 0], idl[q2], semi[q2]).wait()
            pltpu.async_copy(tab.at[isl[q2]], rows[p], semr[p])

    for b in range(4):
        p = b % 2
        pltpu.make_async_copy(tab.at[isl[0]], rows[p], semr[p]).wait()
        pltpu.sync_copy(rows[p], acc_sh.at[idl[b]], add=True)
        if b < 2:
            q2 = b + 2
            pltpu.make_async_copy(src3.at[w, 0], isl[q2], semi[q2]).wait()
            pltpu.make_async_copy(src3.at[w, 0], idl[q2], semi[q2]).wait()
            pltpu.async_copy(tab.at[isl[q2]], rows[p], semr[p])


_EDGE_SCRATCH = (
    [pltpu.VMEM((_K,), jnp.int32)] * 8
    + [pltpu.VMEM((_K, 128), jnp.float32)] * 2
    + [pltpu.SemaphoreType.DMA] * 6
    + [pltpu.VMEM_SHARED((N, 128), jnp.float32)]
)


def _agg_body(src3_hbm, dst3_hbm, hwa_hbm, hwb_hbm, zeros_hbm,
              outa_hbm, outb_hbm, i0, i1, i2, i3, d0, d1, d2, d3,
              r0, r1, si0, si1, si2, si3, sr0, sr1, acc_sh):
    c = lax.axis_index("c")
    s = lax.axis_index("s")

    def work(tab, out):
        _copy_rows(zeros_hbm, acc_sh, s)
        plsc.subcore_barrier()
        _edge_pipeline(tab, acc_sh, src3_hbm, dst3_hbm, s, _NCC,
                       (i0, i1, i2, i3), (d0, d1, d2, d3), (r0, r1),
                       (si0, si1, si2, si3), (sr0, sr1))
        plsc.subcore_barrier()
        _copy_rows(acc_sh, out, s)

    @pl.when(c == 0)
    def _():
        work(hwa_hbm, outa_hbm)

    @pl.when(c == 1)
    def _():
        work(hwb_hbm, outb_hbm)


def _agg(src3, dst3, hwa, hwb, zeros):
    return pl.kernel(
        _agg_body,
        out_type=(jax.ShapeDtypeStruct((N, 128), jnp.float32),
                  jax.ShapeDtypeStruct((N, 128), jnp.float32)),
        mesh=_mesh(),
        scratch_types=list(_EDGE_SCRATCH),
    )(src3, dst3, hwa, hwb, zeros)


def _agg2_body(src3_hbm, dst3_hbm, hw_hbm, zeros_hbm,
               out0_hbm, out1_hbm, i0, i1, i2, i3, d0, d1, d2, d3,
               r0, r1, si0, si1, si2, si3, sr0, sr1, acc_sh):
    c = lax.axis_index("c")
    s = lax.axis_index("s")
    w = c * _TILES + s

    _copy_rows(zeros_hbm, acc_sh, s)
    plsc.subcore_barrier()
    _edge_pipeline(hw_hbm, acc_sh, src3_hbm, dst3_hbm, w, _NCE,
                   (i0, i1, i2, i3), (d0, d1, d2, d3), (r0, r1),
                   (si0, si1, si2, si3), (sr0, sr1))
    plsc.subcore_barrier()

    @pl.when(c == 0)
    def _():
        _copy_rows(acc_sh, out0_hbm, s)

    @pl.when(c == 1)
    def _():
        _copy_rows(acc_sh, out1_hbm, s)


def _agg2(src3, dst3, hw, zeros):
    return pl.kernel(
        _agg2_body,
        out_type=(jax.ShapeDtypeStruct((N, 128), jnp.float32),
                  jax.ShapeDtypeStruct((N, 128), jnp.float32)),
        mesh=_mesh(),
        scratch_types=list(_EDGE_SCRATCH),
    )(src3, dst3, hw, zeros)


def _mm_body(h_ref, w_ref, dis_ref, a_ref, b_ref, *, hh):
    dis = dis_ref[:, 0:1]
    hw = jnp.dot(h_ref[:, :], w_ref[:, :], preferred_element_type=jnp.float32) * dis
    a_ref[:, :] = hw[:, :hh]
    b_ref[:, :] = hw[:, hh:]


def _mm(h, w, deg):
    din = h.shape[1]
    hfull = w.shape[1]
    hh = hfull // 2
    return pl.pallas_call(
        functools.partial(_mm_body, hh=hh),
        grid=(N // _BN,),
        in_specs=[
            pl.BlockSpec((_BN, din), lambda i: (i, 0)),
            pl.BlockSpec((din, hfull), lambda i: (0, 0)),
            pl.BlockSpec((_BN, 16), lambda i: (i, 0)),
        ],
        out_specs=[
            pl.BlockSpec((_BN, hh), lambda i: (i, 0)),
            pl.BlockSpec((_BN, hh), lambda i: (i, 0)),
        ],
        out_shape=[jax.ShapeDtypeStruct((N, hh), jnp.float32)] * 2,
    )(h, w, deg)


def _mmf_body(h_ref, w_ref, dis_ref, o_ref):
    dis = dis_ref[:, 0:1]
    o_ref[:, :] = jnp.dot(h_ref[:, :], w_ref[:, :],
                          preferred_element_type=jnp.float32) * dis


def _mmf(h, w, deg):
    din = h.shape[1]
    hfull = w.shape[1]
    return pl.pallas_call(
        _mmf_body,
        grid=(N // _BN,),
        in_specs=[
            pl.BlockSpec((_BN, din), lambda i: (i, 0)),
            pl.BlockSpec((din, hfull), lambda i: (0, 0)),
            pl.BlockSpec((_BN, 16), lambda i: (i, 0)),
        ],
        out_specs=pl.BlockSpec((_BN, hfull), lambda i: (i, 0)),
        out_shape=jax.ShapeDtypeStruct((N, hfull), jnp.float32),
    )(h, w, deg)


def _post_body(aa_ref, ab_ref, ha_ref, hb_ref, dis_ref, seg_ref,
               h_ref, summ_ref, cnt_ref):
    i = pl.program_id(0)
    dis = dis_ref[:, 0:1]
    left = (aa_ref[:, :] + ha_ref[:, :]) * dis
    right = (ab_ref[:, :] + hb_ref[:, :]) * dis
    h = jnp.maximum(jnp.concatenate([left, right], axis=1), 0.0)
    h_ref[:, :] = h
    onehot = (seg_ref[:, :] == lax.broadcasted_iota(jnp.int32, (_BN, G), 1)
              ).astype(jnp.float32)
    psum = lax.dot_general(onehot, h, (((0,), (0,)), ((), ())),
                           preferred_element_type=jnp.float32)
    pcnt = lax.dot_general(onehot, jnp.ones((_BN, 128), jnp.float32),
                           (((0,), (0,)), ((), ())),
                           preferred_element_type=jnp.float32)

    @pl.when(i == 0)
    def _():
        summ_ref[:, :] = jnp.zeros_like(summ_ref)
        cnt_ref[:, :] = jnp.zeros_like(cnt_ref)

    summ_ref[:, :] += psum
    cnt_ref[:, :] += pcnt


def _post2_body(p0_ref, p1_ref, hw_ref, dis_ref, seg_ref,
                h_ref, summ_ref, cnt_ref):
    i = pl.program_id(0)
    dis = dis_ref[:, 0:1]
    h = jnp.maximum((p0_ref[:, :] + p1_ref[:, :] + hw_ref[:, :]) * dis, 0.0)
    h_ref[:, :] = h
    onehot = (seg_ref[:, :] == lax.broadcasted_iota(jnp.int32, (_BN, G), 1)
              ).astype(jnp.float32)
    psum = lax.dot_general(onehot, h, (((0,), (0,)), ((), ())),
                           preferred_element_type=jnp.float32)
    pcnt = lax.dot_general(onehot, jnp.ones((_BN, 128), jnp.float32),
                           (((0,), (0,)), ((), ())),
                           preferred_element_type=jnp.float32)

    @pl.when(i == 0)
    def _():
        summ_ref[:, :] = jnp.zeros_like(summ_ref)
        cnt_ref[:, :] = jnp.zeros_like(cnt_ref)

    summ_ref[:, :] += psum
    cnt_ref[:, :] += pcnt


def _post2(p0, p1, hw, deg, segr):
    hfull = hw.shape[1]
    return pl.pallas_call(
        _post2_body,
        grid=(N // _BN,),
        in_specs=[
            pl.BlockSpec((_BN, hfull), lambda i: (i, 0)),
            pl.BlockSpec((_BN, hfull), lambda i: (i, 0)),
            pl.BlockSpec((_BN, hfull), lambda i: (i, 0)),
            pl.BlockSpec((_BN, 16), lambda i: (i, 0)),
            pl.BlockSpec((_BN, G), lambda i: (i, 0)),
        ],
        out_specs=[
            pl.BlockSpec((_BN, hfull), lambda i: (i, 0)),
            pl.BlockSpec((G, hfull), lambda i: (0, 0)),
            pl.BlockSpec((G, 128), lambda i: (0, 0)),
        ],
        out_shape=[
            jax.ShapeDtypeStruct((N, hfull), jnp.float32),
            jax.ShapeDtypeStruct((G, hfull), jnp.float32),
            jax.ShapeDtypeStruct((G, 128), jnp.float32),
        ],
    )(p0, p1, hw, deg, segr)


def _post(aa, ab, ha, hb, deg, segr):
    hh = aa.shape[1]
    hfull = 2 * hh
    return pl.pallas_call(
        _post_body,
        grid=(N // _BN,),
        in_specs=[
            pl.BlockSpec((_BN, hh), lambda i: (i, 0)),
            pl.BlockSpec((_BN, hh), lambda i: (i, 0)),
            pl.BlockSpec((_BN, hh), lambda i: (i, 0)),
            pl.BlockSpec((_BN, hh), lambda i: (i, 0)),
            pl.BlockSpec((_BN, 16), lambda i: (i, 0)),
            pl.BlockSpec((_BN, G), lambda i: (i, 0)),
        ],
        out_specs=[
            pl.BlockSpec((_BN, hfull), lambda i: (i, 0)),
            pl.BlockSpec((G, hfull), lambda i: (0, 0)),
            pl.BlockSpec((G, 128), lambda i: (0, 0)),
        ],
        out_shape=[
            jax.ShapeDtypeStruct((N, hfull), jnp.float32),
            jax.ShapeDtypeStruct((G, hfull), jnp.float32),
            jax.ShapeDtypeStruct((G, 128), jnp.float32),
        ],
    )(aa, ab, ha, hb, deg, segr)


def _pool_body(h_ref, seg_ref, summ_ref, cnt_ref, wp_ref, g_ref):
    i = pl.program_id(0)
    mean = summ_ref[:, :] / jnp.maximum(cnt_ref[:, 0:1], 1.0)
    cmat = jnp.tanh(jnp.dot(mean, wp_ref[:, :],
                            preferred_element_type=jnp.float32))
    onehot = (seg_ref[:, :] == lax.broadcasted_iota(jnp.int32, (_BN, G), 1)
              ).astype(jnp.float32)
    cnode = jnp.dot(onehot, cmat, preferred_element_type=jnp.float32)
    h = h_ref[:, :]
    att = 1.0 / (1.0 + jnp.exp(-jnp.sum(h * cnode, axis=1, keepdims=True)))

    @pl.when(i == 0)
    def _():
        g_ref[:, :] = jnp.zeros_like(g_ref)

    g_ref[:, :] += lax.dot_general(onehot, att * h, (((0,), (0,)), ((), ())),
                                   preferred_element_type=jnp.float32)


def _pool(h, segr, summ, cnt, wp):
    hfull = h.shape[1]
    return pl.pallas_call(
        _pool_body,
        grid=(N // _BN,),
        in_specs=[
            pl.BlockSpec((_BN, hfull), lambda i: (i, 0)),
            pl.BlockSpec((_BN, G), lambda i: (i, 0)),
            pl.BlockSpec((G, hfull), lambda i: (0, 0)),
            pl.BlockSpec((G, 128), lambda i: (0, 0)),
            pl.BlockSpec((hfull, hfull), lambda i: (0, 0)),
        ],
        out_specs=pl.BlockSpec((G, hfull), lambda i: (0, 0)),
        out_shape=jax.ShapeDtypeStruct((G, hfull), jnp.float32),
    )(h, segr, summ, cnt, wp)


def _head_body(g1_ref, g2_ref, g3_ref, w1a_ref, w1b_ref, w1c_ref, b1_ref,
               w2_ref, b2_ref, w3_ref, b3_ref, wd1_ref, bd1_ref,
               wd2_ref, bd2_ref, out_ref):
    m = (jnp.dot(g1_ref[:, :], w1a_ref[:, :], preferred_element_type=jnp.float32)
         + jnp.dot(g2_ref[:, :], w1b_ref[:, :], preferred_element_type=jnp.float32)
         + jnp.dot(g3_ref[:, :], w1c_ref[:, :], preferred_element_type=jnp.float32)
         + b1_ref[:, :])
    m = jnp.maximum(m, 0.0)
    m = jnp.maximum(jnp.dot(m, w2_ref[:, :], preferred_element_type=jnp.float32)
                    + b2_ref[:, :], 0.0)
    emb = jnp.dot(m, w3_ref[:, :], preferred_element_type=jnp.float32) + b3_ref[:, :]
    d = jnp.maximum(jnp.dot(emb, wd1_ref[:, :], preferred_element_type=jnp.float32)
                    + bd1_ref[:, :], 0.0)
    out_ref[:, :] = 0.5 * jnp.tanh(
        jnp.dot(d, wd2_ref[:, :], preferred_element_type=jnp.float32)
        + bd2_ref[:, :])


def _head(g1, g2, g3, w1a, w1b, w1c, bm1, wm2, bm2, wm3, bm3,
          wd1, bd1, wd2, bd2):
    return pl.pallas_call(
        _head_body,
        out_shape=jax.ShapeDtypeStruct((G, wd2.shape[1]), jnp.float32),
    )(g1, g2, g3, w1a, w1b, w1c,
      bm1[None, :], wm2, bm2[None, :], wm3, bm3[None, :],
      wd1, bd1[None, :], wd2, bd2[None, :])


def kernel(x, edge_index, segment_ids, W1, Wp1, W2, Wp2, W3, Wp3,
           Wm1, bm1, Wm2, bm2, Wm3, bm3, Wd1, bd1, Wd2, bd2):
    src = edge_index[0].astype(jnp.int32)
    dst = edge_index[1].astype(jnp.int32)
    src_cs = src.reshape(_TILES, _NCC, _K)
    dst_cs = dst.reshape(_TILES, _NCC, _K)
    src_es = src.reshape(2 * _TILES, _NCE, _K)
    dst_es = dst.reshape(2 * _TILES, _NCE, _K)
    segr = jnp.broadcast_to(segment_ids.astype(jnp.int32)[:, None], (N, G))

    zeros128 = jnp.zeros((N, 128), jnp.float32)
    ones128 = jnp.ones((_K, 128), jnp.float32)

    W3p = jnp.concatenate([W3, jnp.zeros((W3.shape[0], 64), jnp.float32)], 1)
    Wp3p = jnp.pad(Wp3, ((0, 64), (0, 64)))
    w1c = jnp.concatenate([Wm1[384:], jnp.zeros((64, Wm1.shape[1]),
                                                jnp.float32)], 0)

    d0, d1 = _deg(dst_es, zeros128, ones128)
    deg = _dis(d0, d1)

    hwa, hwb = _mm(x, W1, deg)
    aga, agb = _agg(src_cs, dst_cs, hwa, hwb, zeros128)
    h1, summ1, cnt = _post(aga, agb, hwa, hwb, deg, segr)
    g1 = _pool(h1, segr, summ1, cnt, Wp1)

    hw2 = _mmf(h1, W2, deg)
    p0, p1 = _agg2(src_es, dst_es, hw2, zeros128)
    h2, summ2, cnt2 = _post2(p0, p1, hw2, deg, segr)
    g2 = _pool(h2, segr, summ2, cnt2, Wp2)

    hw3 = _mmf(h2, W3p, deg)
    p0, p1 = _agg2(src_es, dst_es, hw3, zeros128)
    h3, summ3, cnt3 = _post2(p0, p1, hw3, deg, segr)
    g3 = _pool(h3, segr, summ3, cnt3, Wp3p)

    return _head(g1, g2, g3, Wm1[:256], Wm1[256:384], w1c, bm1,
                 Wm2, bm2, Wm3, bm3, Wd1, bd1, Wd2, bd2)
